# baseline (speedup 1.0000x reference)
import dataclasses
import functools

import jax
import jax.numpy as jnp
from jax import lax
from jax.experimental import pallas as pl
from jax.experimental.pallas import tpu as pltpu
from jax.experimental.pallas import tpu_sc as plsc

N = 320000
D = 128
S = 10000
WS = 16
NWIN = S // WS
NW = 32
C = 128
LANES = 16
NSL = D // LANES
BIG = 3.0e38
TBL = 352
BT_PAD = 10048

_mesh = plsc.VectorSubcoreMesh(core_axis_name="c", subcore_axis_name="s")

_cp = pltpu.CompilerParams()
if "needs_layout_passes" in pltpu.CompilerParams.__dataclass_fields__:
    _cp = dataclasses.replace(_cp, needs_layout_passes=False)


@functools.partial(
    pl.kernel,
    out_type=jax.ShapeDtypeStruct((S, 3 * D), jnp.float32),
    mesh=_mesh,
    compiler_params=_cp,
    scratch_types=[
        pltpu.VMEM((TBL,), jnp.int32),
        pltpu.VMEM((C, D), jnp.float32),
        pltpu.VMEM((C, D), jnp.float32),
        pltpu.VMEM((3 * D,), jnp.float32),
        pltpu.VMEM((WS, 3 * D), jnp.float32),
        pltpu.VMEM((WS, 3 * D), jnp.float32),
        pltpu.SemaphoreType.DMA,
        pltpu.SemaphoreType.DMA,
        pltpu.SemaphoreType.DMA,
        pltpu.SemaphoreType.DMA,
    ],
)
def _sc_aggr(x_hbm, bt_hbm, out_hbm, tbl, xbuf0, xbuf1, run, acc0, acc1,
             semx0, semx1, semo0, semo1):
    cid = lax.axis_index("c")
    sid = lax.axis_index("s")
    wid = sid * 2 + cid
    xbufs = (xbuf0, xbuf1)
    xsems = (semx0, semx1)

    w0 = (wid * NWIN) // NW
    w1 = ((wid + 1) * NWIN) // NW
    nw_local = w1 - w0
    s_lo = w0 * WS

    pltpu.sync_copy(bt_hbm.at[pl.ds(s_lo, TBL)], tbl)

    zerov = jnp.zeros((LANES,), jnp.float32)
    negbig = zerov - BIG
    posbig = zerov + BIG

    def start_dma(m, b):
        off = pl.multiple_of(m * C, 8)
        pltpu.async_copy(x_hbm.at[pl.ds(off, C), :], xbufs[b], xsems[b])

    def wait_dma(b):
        pltpu.make_async_copy(x_hbm.at[pl.ds(0, C), :], xbufs[b],
                              xsems[b]).wait()

    def row_loop(xb, lo, hi, init):
        def body(q, regs):
            vs = [xb[q, pl.ds(c * LANES, LANES)] for c in range(NSL)]
            new = []
            for c in range(NSL):
                new.append(regs[c] + vs[c])
            for c in range(NSL):
                new.append(jnp.maximum(regs[NSL + c], vs[c]))
            for c in range(NSL):
                new.append(jnp.minimum(regs[2 * NSL + c], vs[c]))
            return tuple(new)
        return lax.fori_loop(lo, hi, body, init)

    def win_body(w, acc, osem):
        wbase = w * WS
        loff = wbase - s_lo

        @pl.when(w - w0 >= 2)
        def _():
            pltpu.make_async_copy(acc, out_hbm.at[pl.ds(0, WS), :],
                                  osem).wait()

        for l in range(WS):
            for c in range(3 * NSL):
                acc[l, pl.ds(c * LANES, LANES)] = zerov

        bv0 = tbl[pl.ds(loff, LANES)]
        bv1 = tbl[pl.ds(loff + WS, LANES)]
        rs = bv0[0]
        re = bv1[0]

        m0 = rs // C
        m1 = (re + (C - 1)) // C

        @pl.when(m0 < m1)
        def _():
            start_dma(m0, 0)

        def flush(p, regs, cnt):
            rec = jnp.broadcast_to(jnp.float32(1.0), (LANES,)) / \
                jnp.broadcast_to(cnt.astype(jnp.float32), (LANES,))
            for c in range(NSL):
                acc[p, pl.ds(c * LANES, LANES)] = regs[c] * rec
                acc[p, pl.ds(D + c * LANES, LANES)] = regs[NSL + c]
                acc[p, pl.ds(2 * D + c * LANES, LANES)] = regs[2 * NSL + c]

        def piece_factory(b, base):
            xb = xbufs[b]

            def piece(p, carry):
                bv = tbl[pl.ds(loff + p, LANES)]
                r0 = bv[0]
                r1 = bv[1]
                start = jnp.maximum(r0, base)
                end = jnp.minimum(r1, base + C)
                active = start < end
                fresh = r0 >= base
                ended = r1 <= base + C
                freshv = jnp.broadcast_to(fresh, (LANES,))

                ident = ((zerov,) * NSL + (negbig,) * NSL +
                         (posbig,) * NSL)
                init = tuple(
                    jnp.where(freshv, ident[c], run[pl.ds(c * LANES,
                                                          LANES)])
                    for c in range(3 * NSL))
                qlo = jnp.maximum(start - base, 0)
                qhi = jnp.maximum(end - base, qlo)
                regs = row_loop(xb, qlo, qhi, init)

                @pl.when(active & ended)
                def _():
                    flush(p, regs, r1 - r0)

                @pl.when(active & jnp.logical_not(ended))
                def _():
                    for c in range(3 * NSL):
                        run[pl.ds(c * LANES, LANES)] = regs[c]

                return carry
            return piece

        def process(m, b):
            @pl.when(m < m1)
            def _():
                wait_dma(b)

                @pl.when(m + 1 < m1)
                def _():
                    start_dma(m + 1, 1 - b)

            lax.fori_loop(0, WS, piece_factory(b, m * C), 0)

        npair = (m1 - m0 + 1) // 2

        def pair_body(i, carry):
            m = m0 + 2 * i
            process(m, 0)
            process(m + 1, 1)
            return carry

        lax.fori_loop(0, npair, pair_body, 0)

        pltpu.async_copy(acc, out_hbm.at[pl.ds(wbase, WS), :], osem)

    def win_pair(j, carry):
        w = w0 + 2 * j

        @pl.when(w < w1)
        def _():
            win_body(w, acc0, semo0)

        @pl.when(w + 1 < w1)
        def _():
            win_body(w + 1, acc1, semo1)

        return carry

    lax.fori_loop(0, (nw_local + 1) // 2, win_pair, 0)

    @pl.when(nw_local >= 1)
    def _():
        pltpu.make_async_copy(acc0, out_hbm.at[pl.ds(0, WS), :],
                              semo0).wait()

    @pl.when(nw_local >= 2)
    def _():
        pltpu.make_async_copy(acc1, out_hbm.at[pl.ds(0, WS), :],
                              semo1).wait()


def kernel(x, batch):
    b32 = batch.astype(jnp.int32)
    bounds = jnp.arange(S + 1, dtype=jnp.int32)
    bt = jnp.searchsorted(b32, bounds).astype(jnp.int32)
    bt_pad = jnp.concatenate(
        [bt, jnp.full((BT_PAD - (S + 1),), N, jnp.int32)])
    return _sc_aggr(x, bt_pad)


# speedup vs baseline: 11.4743x; 11.4743x over previous
import dataclasses
import functools

import jax
import jax.numpy as jnp
from jax import lax
from jax.experimental import pallas as pl
from jax.experimental.pallas import tpu as pltpu
from jax.experimental.pallas import tpu_sc as plsc

N = 320000
D = 128
S = 10000
WS = 16
NWIN = S // WS
NW = 32
C = 128
LANES = 16
NSL = D // LANES
BIG = 3.0e38

_mesh = plsc.VectorSubcoreMesh(core_axis_name="c", subcore_axis_name="s")

_cp = pltpu.CompilerParams()
if "needs_layout_passes" in pltpu.CompilerParams.__dataclass_fields__:
    _cp = dataclasses.replace(_cp, needs_layout_passes=False)


@functools.partial(
    pl.kernel,
    out_type=jax.ShapeDtypeStruct((S, 3 * D), jnp.float32),
    mesh=_mesh,
    compiler_params=_cp,
    scratch_types=[
        pltpu.VMEM((NWIN + 31,), jnp.int32),
        pltpu.VMEM((C, D), jnp.float32),
        pltpu.VMEM((C, D), jnp.float32),
        pltpu.VMEM((2 * C,), jnp.int32),
        pltpu.VMEM((2 * C,), jnp.int32),
        pltpu.VMEM((WS, 3 * D), jnp.float32),
        pltpu.SemaphoreType.DMA,
        pltpu.SemaphoreType.DMA,
    ],
)
def _sc_aggr(x_hbm, b_hbm, ws_hbm, out_hbm, ws_v, xbuf0, xbuf1, bbuf0,
             bbuf1, acc, sem0, sem1):
    cid = lax.axis_index("c")
    sid = lax.axis_index("s")
    wid = sid * 2 + cid
    sems = (sem0, sem1)
    xbufs = (xbuf0, xbuf1)
    bbufs = (bbuf0, bbuf1)

    pltpu.sync_copy(ws_hbm, ws_v)

    w0 = (wid * NWIN) // NW
    w1 = ((wid + 1) * NWIN) // NW

    zerov = jnp.zeros((LANES,), jnp.float32)
    negbig = zerov - BIG
    posbig = zerov + BIG

    def start_dma(m, b):
        off = pl.multiple_of(jnp.minimum(m * C, N - C), 8)
        pltpu.async_copy(x_hbm.at[pl.ds(off, C), :], xbufs[b], sems[b])
        pltpu.async_copy(b_hbm.at[pl.ds(off, C)],
                         bbufs[b].at[pl.ds(0, C)], sems[b])

    def wait_dma(b):
        pltpu.make_async_copy(x_hbm.at[pl.ds(0, C), :], xbufs[b],
                              sems[b]).wait()
        pltpu.make_async_copy(b_hbm.at[pl.ds(0, C)],
                              bbufs[b].at[pl.ds(0, C)], sems[b]).wait()

    def win_body(w, _):
        wbase = w * WS
        wsv = ws_v[pl.ds(w, LANES)]
        rs = wsv[0]
        re = wsv[1]

        for l in range(WS):
            for c in range(3 * NSL):
                acc[l, pl.ds(c * LANES, LANES)] = zerov

        m0 = rs // C
        m1 = (re + (C - 1)) // C
        nch = m1 - m0
        npair = (nch + 1) // 2

        @pl.when(m0 < m1)
        def _():
            start_dma(m0, 0)

        @pl.when(m0 + 1 < m1)
        def _():
            start_dma(m0 + 1, 1)

        def flush(lc, cr, regs):
            cntv = jnp.broadcast_to(cr, (LANES,))
            for c in range(NSL):
                acc[lc, pl.ds(c * LANES, LANES)] = regs[c] / cntv
                acc[lc, pl.ds(D + c * LANES, LANES)] = regs[NSL + c]
                acc[lc, pl.ds(2 * D + c * LANES, LANES)] = regs[2 * NSL + c]

        def make_row_body(b):
            xb = xbufs[b]
            bb = bbufs[b]

            def row_body(q, carry):
                lc, cr = carry[0], carry[1]
                regs = carry[2:]
                l_row = bb[pl.ds(q, LANES)][0] - wbase
                changed = l_row != lc

                @pl.when(changed & (lc >= 0))
                def _():
                    flush(lc, cr, regs)

                chv = jnp.broadcast_to(changed, (LANES,))
                kv = jnp.where(chv, 0.0, 1.0)
                new = [l_row, jnp.where(changed, 1.0, cr + 1.0)]
                for c in range(NSL):
                    v = xb[q, pl.ds(c * LANES, LANES)]
                    new.append(regs[c] * kv + v)
                for c in range(NSL):
                    v = xb[q, pl.ds(c * LANES, LANES)]
                    new.append(jnp.maximum(
                        jnp.where(chv, negbig, regs[NSL + c]), v))
                for c in range(NSL):
                    v = xb[q, pl.ds(c * LANES, LANES)]
                    new.append(jnp.minimum(
                        jnp.where(chv, posbig, regs[2 * NSL + c]), v))
                return tuple(new)
            return row_body

        row_bodies = (make_row_body(0), make_row_body(1))

        def process(m, b, carry):
            @pl.when(m < m1)
            def _():
                wait_dma(b)

            base = m * C
            lo = jnp.maximum(rs - base, 0)
            hi = jnp.minimum(re - base, C)
            carry = plsc.parallel_loop(lo, hi, carry=carry)(row_bodies[b])

            @pl.when(m + 2 < m1)
            def _():
                start_dma(m + 2, b)

            return carry

        init = (jnp.int32(-1), jnp.float32(0.0)) + (zerov,) * (3 * NSL)

        def pair_body(i, carry):
            m = m0 + 2 * i
            carry = process(m, 0, carry)
            carry = process(m + 1, 1, carry)
            return carry

        carry = lax.fori_loop(0, npair, pair_body, init)

        lc, cr = carry[0], carry[1]

        @pl.when(lc >= 0)
        def _():
            flush(lc, cr, carry[2:])

        pltpu.sync_copy(acc, out_hbm.at[pl.ds(wbase, WS), :])
        return 0

    lax.fori_loop(w0, w1, win_body, 0)


def kernel(x, batch):
    b32 = batch.astype(jnp.int32)
    bounds = jnp.arange(NWIN + 1, dtype=jnp.int32) * WS
    ws = jnp.searchsorted(b32, bounds).astype(jnp.int32)
    ws_pad = jnp.concatenate([ws, jnp.full((30,), N, jnp.int32)])
    return _sc_aggr(x, b32, ws_pad)
